# baseline (speedup 1.0000x reference)
import jax
import jax.numpy as jnp
from jax import lax
from jax.experimental import pallas as pl
from jax.experimental.pallas import tpu as pltpu

EPS = 1e-5
NB = 10

_DN = (((1,), (1,)), ((), ()))


def _bn_coeffs(s, q, n, gamma, beta):
    inv_n = jnp.float32(1.0 / n)
    mean = s * inv_n
    var = q * inv_n - mean * mean
    scale = gamma * lax.rsqrt(var + EPS)
    return scale, beta - mean * scale


def _mlp_kernel(x_hbm, w0_ref, w1_ref, w2_ref, b2_ref, gamma_ref, beta_ref,
                out_hbm, xb, ob, h_ref, in_sem, out_sem):
    n = x_hbm.shape[0]
    br = n // NB
    gamma = gamma_ref[...]
    beta = beta_ref[...]
    w0 = w0_ref[...]

    in_copies = [
        pltpu.make_async_copy(x_hbm.at[pl.ds(b * br, br), :], xb.at[b % 2],
                              in_sem.at[b % 2])
        for b in range(NB)
    ]
    in_copies[0].start()
    in_copies[1].start()

    s = None
    for b in range(NB):
        in_copies[b].wait()
        hb = lax.dot_general(xb[b % 2], w0, _DN,
                             preferred_element_type=jnp.float32)
        if b + 2 < NB:
            in_copies[b + 2].start()
        h_ref[pl.ds(b * br, br), :] = hb
        sb = jnp.sum(hb, axis=0, keepdims=True)
        qb = jnp.sum(hb * hb, axis=0, keepdims=True)
        s = sb if s is None else s + sb
        q = qb if b == 0 else q + qb

    scale, shift = _bn_coeffs(s, q, n, gamma, beta)
    t = jnp.tanh(h_ref[...] * scale + shift)
    h1 = lax.dot_general(t, w1_ref[...], _DN,
                         preferred_element_type=jnp.float32)
    s1 = jnp.sum(h1, axis=0, keepdims=True)
    q1 = jnp.sum(h1 * h1, axis=0, keepdims=True)
    h_ref[...] = h1
    scale, shift = _bn_coeffs(s1, q1, n, gamma, beta)

    w2 = w2_ref[...]
    b2 = b2_ref[...]
    out_copies = [
        pltpu.make_async_copy(ob.at[b % 2], out_hbm.at[pl.ds(b * br, br), :],
                              out_sem.at[b % 2])
        for b in range(NB)
    ]
    for b in range(NB):
        t2 = jnp.tanh(h_ref[pl.ds(b * br, br), :] * scale + shift)
        if b >= 2:
            out_copies[b - 2].wait()
        ob[b % 2] = lax.dot_general(t2, w2, _DN,
                                    preferred_element_type=jnp.float32) + b2
        out_copies[b].start()
    out_copies[NB - 2].wait()
    out_copies[NB - 1].wait()


def kernel(x, edge_index, edge_attr, W0, b0, W1, b1, W2, b2, gamma, beta):
    del edge_index, edge_attr
    del b0, b1
    n, d_in = x.shape
    d_h = W0.shape[0]
    d_out = W2.shape[0]
    br = n // NB
    vmem = pl.BlockSpec(memory_space=pltpu.VMEM)
    hbm = pl.BlockSpec(memory_space=pl.ANY)
    return pl.pallas_call(
        _mlp_kernel,
        in_specs=[hbm, vmem, vmem, vmem, vmem, vmem, vmem],
        out_specs=hbm,
        out_shape=jax.ShapeDtypeStruct((n, d_out), jnp.float32),
        scratch_shapes=[
            pltpu.VMEM((2, br, d_in), jnp.float32),
            pltpu.VMEM((2, br, d_out), jnp.float32),
            pltpu.VMEM((n, d_h), jnp.float32),
            pltpu.SemaphoreType.DMA((2,)),
            pltpu.SemaphoreType.DMA((2,)),
        ],
    )(x, W0, W1, W2, b2[None, :], gamma[None, :], beta[None, :])


# speedup vs baseline: 1.0059x; 1.0059x over previous
import jax
import jax.numpy as jnp
from jax import lax
from jax.experimental import pallas as pl
from jax.experimental.pallas import tpu as pltpu

EPS = 1e-5
NB = 10

_DN = (((1,), (1,)), ((), ()))


def _bn_coeffs(s, q, n, gamma, beta):
    inv_n = jnp.float32(1.0 / n)
    mean = s * inv_n
    var = q * inv_n - mean * mean
    scale = gamma * lax.rsqrt(var + EPS)
    return scale, beta - mean * scale


def _mlp_kernel(x_hbm, w0_ref, w1_ref, w2_ref, b2_ref, gamma_ref, beta_ref,
                out_hbm, h_ref, s_ref, q_ref):
    n, d_in = x_hbm.shape
    d_h = h_ref.shape[1]
    br = n // NB
    gamma = gamma_ref[...]
    beta = beta_ref[...]
    w0 = w0_ref[...]

    s_ref[...] = jnp.zeros_like(s_ref)
    q_ref[...] = jnp.zeros_like(q_ref)

    def l0_body(idx, x_blk):
        b = idx[0]
        hb = lax.dot_general(x_blk[...], w0, _DN,
                             preferred_element_type=jnp.float32)
        h_ref[pl.ds(b * br, br), :] = hb
        s_ref[...] += jnp.sum(hb, axis=0)
        q_ref[...] += jnp.sum(hb * hb, axis=0)

    pltpu.emit_pipeline(
        l0_body,
        grid=(NB,),
        in_specs=[pl.BlockSpec((br, d_in), lambda b: (b, 0))],
        _explicit_indices=True,
    )(x_hbm)

    scale, shift = _bn_coeffs(s_ref[...], q_ref[...], n, gamma, beta)
    t = jnp.tanh(h_ref[...] * scale + shift)
    h1 = lax.dot_general(t, w1_ref[...], _DN,
                         preferred_element_type=jnp.float32)
    s1 = jnp.sum(h1, axis=0)
    q1 = jnp.sum(h1 * h1, axis=0)
    h_ref[...] = h1
    scale, shift = _bn_coeffs(s1, q1, n, gamma, beta)

    w2 = w2_ref[...]
    b2 = b2_ref[...]

    def l2_body(idx, o_blk):
        b = idx[0]
        t2 = jnp.tanh(h_ref[pl.ds(b * br, br), :] * scale + shift)
        o_blk[...] = lax.dot_general(t2, w2, _DN,
                                     preferred_element_type=jnp.float32) + b2

    pltpu.emit_pipeline(
        l2_body,
        grid=(NB,),
        out_specs=[pl.BlockSpec((br, d_h), lambda b: (b, 0))],
        _explicit_indices=True,
    )(out_hbm)


def kernel(x, edge_index, edge_attr, W0, b0, W1, b1, W2, b2, gamma, beta):
    del edge_index, edge_attr
    del b0, b1
    n, _ = x.shape
    d_h = W0.shape[0]
    d_out = W2.shape[0]
    vmem = pl.BlockSpec(memory_space=pltpu.VMEM)
    hbm = pl.BlockSpec(memory_space=pl.ANY)
    return pl.pallas_call(
        _mlp_kernel,
        in_specs=[hbm, vmem, vmem, vmem, vmem, vmem, vmem],
        out_specs=hbm,
        out_shape=jax.ShapeDtypeStruct((n, d_out), jnp.float32),
        scratch_shapes=[
            pltpu.VMEM((n, d_h), jnp.float32),
            pltpu.VMEM((d_h,), jnp.float32),
            pltpu.VMEM((d_h,), jnp.float32),
        ],
    )(x, W0, W1, W2, b2, gamma, beta)


# speedup vs baseline: 1.7281x; 1.7179x over previous
import jax
import jax.numpy as jnp
from jax import lax
from jax.experimental import pallas as pl

EPS = 1e-5

_DN = (((1,), (1,)), ((), ()))


def _bn_tanh(h, n, gamma, beta):
    inv_n = jnp.float32(1.0 / n)
    s = jnp.sum(h, axis=0)
    q = jnp.sum(h * h, axis=0)
    mean = s * inv_n
    var = q * inv_n - mean * mean
    scale = gamma * lax.rsqrt(var + EPS)
    shift = beta - mean * scale
    return jnp.tanh(h * scale + shift)


def _mlp_kernel(x_ref, w0_ref, w1_ref, w2_ref, b2_ref, gamma_ref, beta_ref,
                out_ref):
    n = x_ref.shape[0]
    gamma = gamma_ref[...]
    beta = beta_ref[...]

    h = lax.dot_general(x_ref[...], w0_ref[...], _DN,
                        preferred_element_type=jnp.float32)
    h = _bn_tanh(h, n, gamma, beta)
    h = lax.dot_general(h, w1_ref[...], _DN,
                        preferred_element_type=jnp.float32)
    h = _bn_tanh(h, n, gamma, beta)
    h = lax.dot_general(h, w2_ref[...], _DN,
                        preferred_element_type=jnp.float32)
    out_ref[...] = h + b2_ref[...]


def kernel(x, edge_index, edge_attr, W0, b0, W1, b1, W2, b2, gamma, beta):
    del edge_index, edge_attr
    del b0, b1
    n, _ = x.shape
    d_out = W2.shape[0]
    return pl.pallas_call(
        _mlp_kernel,
        out_shape=jax.ShapeDtypeStruct((n, d_out), jnp.float32),
    )(x, W0, W1, W2, b2, gamma, beta)
